# Initial kernel scaffold; baseline (speedup 1.0000x reference)
import functools

import jax
import jax.numpy as jnp
from jax import lax
from jax.experimental import pallas as pl
from jax.experimental.pallas import tpu as pltpu
from jax.experimental.pallas import tpu_sc as plsc

D = 1024
FF = 4096
E = 8
K = 2
AUX_COEF = 0.01
Z_COEF = 0.001

N_TOK = 8192
A = N_TOK * K
T = 512
NT = A // T + E
NT_PAD = 48
A_PAD = NT * T

TB = 1024
FFC = 512
NF = FF // FFC

DSP_W = 16
DSP_CHUNK = A // DSP_W
DSP_VECS = DSP_CHUNK // 16
DSP_SLICE = A_PAD // DSP_W

GW = 32
G_ROWS = A_PAD // GW
G_CHUNK = 64
C_TOK = N_TOK // GW
C_CHUNK = 32


def _router_body(x_ref, wr_ref, idx_ref, w_ref, loss_ref, accp, accc, acclse):
    i = pl.program_id(0)
    x = x_ref[...]
    logits = lax.dot_general(x, wr_ref[...], (((1,), (1,)), ((), ())),
                             preferred_element_type=jnp.float32)
    eidx = lax.broadcasted_iota(jnp.int32, logits.shape, 1)
    m0 = jnp.max(logits, axis=-1, keepdims=True)
    i0 = jnp.min(jnp.where(logits == m0, eidx, E), axis=-1, keepdims=True)
    masked = jnp.where(eidx == i0, -jnp.inf, logits)
    m1 = jnp.max(masked, axis=-1, keepdims=True)
    i1 = jnp.min(jnp.where(masked == m1, eidx, E), axis=-1, keepdims=True)
    e1 = jnp.exp(m1 - m0)
    denom = 1.0 + e1
    idx_ref[...] = jnp.concatenate([i0, i1], axis=1)
    w_ref[...] = jnp.concatenate([1.0 / denom, e1 / denom], axis=1)
    ex = jnp.exp(logits - m0)
    sex = jnp.sum(ex, axis=-1, keepdims=True)
    probs = ex / sex
    lse = m0 + jnp.log(sex)

    @pl.when(i == 0)
    def _():
        accp[...] = jnp.zeros_like(accp)
        accc[...] = jnp.zeros_like(accc)
        acclse[...] = jnp.zeros_like(acclse)

    accp[...] += jnp.sum(probs, axis=0, keepdims=True)
    onehot = jnp.where(eidx == i0, 1.0, 0.0) + jnp.where(eidx == i1, 1.0, 0.0)
    accc[...] += jnp.sum(onehot, axis=0, keepdims=True)
    acclse[...] += jnp.sum(lse).reshape(1, 1)

    @pl.when(i == pl.num_programs(0) - 1)
    def _():
        tokens_per_expert = accc[...] / (N_TOK * K)
        router_prob = accp[...] / N_TOK
        aux = E * jnp.sum(tokens_per_expert * router_prob) * AUX_COEF
        z = acclse[0, 0] / N_TOK * Z_COEF
        loss_ref[0, 0] = aux + z


def _router(x, w_router):
    return pl.pallas_call(
        _router_body,
        grid=(N_TOK // TB,),
        in_specs=[
            pl.BlockSpec((TB, D), lambda i: (i, 0)),
            pl.BlockSpec((E, D), lambda i: (0, 0)),
        ],
        out_specs=[
            pl.BlockSpec((TB, K), lambda i: (i, 0)),
            pl.BlockSpec((TB, K), lambda i: (i, 0)),
            pl.BlockSpec((1, 1), lambda i: (0, 0)),
        ],
        out_shape=[
            jax.ShapeDtypeStruct((N_TOK, K), jnp.int32),
            jax.ShapeDtypeStruct((N_TOK, K), jnp.float32),
            jax.ShapeDtypeStruct((1, 1), jnp.float32),
        ],
        scratch_shapes=[
            pltpu.VMEM((1, E), jnp.float32),
            pltpu.VMEM((1, E), jnp.float32),
            pltpu.VMEM((1, 1), jnp.float32),
        ],
    )(x, w_router)


def _dispatch_body(idx_hbm, w_hbm, pos_hbm, src_hbm, wsort_hbm, te_hbm,
                   idx_c, w_c, pbuf, tbuf, cnt_ref, hist_local, zb_i, zb_f,
                   tebuf, sh_hist, sh_src, sh_w):
    wid = lax.axis_index("s")
    lane = lax.broadcasted_iota(jnp.int32, (16,), 0)
    zi16 = jnp.zeros((16,), jnp.int32)

    def zbody(j, _):
        zb_i[pl.ds(j * 16, 16)] = zi16
        zb_f[pl.ds(j * 16, 16)] = jnp.zeros((16,), jnp.float32)
        return 0
    lax.fori_loop(0, DSP_SLICE // 16, zbody, 0)
    pltpu.sync_copy(zb_i, sh_src.at[pl.ds(wid * DSP_SLICE, DSP_SLICE)])
    pltpu.sync_copy(zb_f, sh_w.at[pl.ds(wid * DSP_SLICE, DSP_SLICE)])

    pltpu.sync_copy(idx_hbm.at[pl.ds(wid * DSP_CHUNK, DSP_CHUNK)], idx_c)
    pltpu.sync_copy(w_hbm.at[pl.ds(wid * DSP_CHUNK, DSP_CHUNK)], w_c)

    def hbody(j, cnt):
        v = idx_c[pl.ds(j * 16, 16)]
        for e in range(E):
            pop = plsc.all_reduce_population_count(v == e)
            cnt = cnt + jnp.where(lane == e, pop, 0)
        return cnt
    local_cnt = lax.fori_loop(0, DSP_VECS, hbody, zi16)
    cnt_ref[...] = local_cnt
    pltpu.sync_copy(cnt_ref, sh_hist.at[pl.ds(wid * 16, 16)])
    plsc.subcore_barrier()

    pltpu.sync_copy(sh_hist, hist_local)

    def rbody(w, carry):
        g, mb = carry
        row = hist_local[pl.ds(w * 16, 16)]
        return g + row, mb + jnp.where(w < wid, row, 0)
    gcnt, my_base = lax.fori_loop(0, DSP_W, rbody, (zi16, zi16))

    padded = (gcnt + (T - 1)) & (-T)
    offs = plsc.cumsum(padded) - padded
    base0 = offs + my_base

    def pbody(j, cnt):
        off = j * 16
        v = idx_c[pl.ds(off, 16)]
        cnt_ref[...] = cnt
        base = plsc.load_gather(cnt_ref, [v])
        rank = zi16
        newcnt = cnt
        for e in range(E):
            m = v == e
            mi = m.astype(jnp.int32)
            cs = plsc.cumsum(mi)
            rank = rank + jnp.where(m, cs - mi, 0)
            pop = plsc.all_reduce_population_count(m)
            newcnt = newcnt + jnp.where(lane == e, pop, 0)
        pbuf[pl.ds(off, 16)] = base + rank
        tbuf[pl.ds(off, 16)] = lax.shift_right_logical(
            wid * DSP_CHUNK + off + lane, 1)
        return newcnt
    lax.fori_loop(0, DSP_VECS, pbody, base0)

    pltpu.sync_copy(pbuf, pos_hbm.at[pl.ds(wid * DSP_CHUNK, DSP_CHUNK)])

    def sbody(j, _):
        off = j * 16
        idxv = pbuf[pl.ds(off, 16)]
        pltpu.sync_copy(tbuf.at[pl.ds(off, 16)], sh_src.at[idxv])
        pltpu.sync_copy(w_c.at[pl.ds(off, 16)], sh_w.at[idxv])
        return 0
    lax.fori_loop(0, DSP_VECS, sbody, 0)
    plsc.subcore_barrier()

    sl = pl.ds(wid * DSP_SLICE, DSP_SLICE)
    pltpu.sync_copy(sh_src.at[sl], src_hbm.at[sl])
    pltpu.sync_copy(sh_w.at[sl], wsort_hbm.at[sl])

    @pl.when(wid == 0)
    def _():
        cnt_ref[...] = offs + padded
        for c in range(NT_PAD // 16):
            tid = c * 16 + lane
            acc = zi16
            for e in range(E):
                acc = acc + jnp.where(tid * T >= cnt_ref[e], 1, 0)
            tebuf[pl.ds(c * 16, 16)] = jnp.minimum(acc, E - 1)
        pltpu.sync_copy(tebuf, te_hbm)


def _dispatch(idx_flat, w_flat):
    mesh = plsc.VectorSubcoreMesh(core_axis_name="c", subcore_axis_name="s",
                                  num_cores=1)
    return pl.kernel(
        _dispatch_body,
        out_type=[
            jax.ShapeDtypeStruct((A,), jnp.int32),
            jax.ShapeDtypeStruct((A_PAD,), jnp.int32),
            jax.ShapeDtypeStruct((A_PAD,), jnp.float32),
            jax.ShapeDtypeStruct((NT_PAD,), jnp.int32),
        ],
        mesh=mesh,
        scratch_types=[
            pltpu.VMEM((DSP_CHUNK,), jnp.int32),
            pltpu.VMEM((DSP_CHUNK,), jnp.float32),
            pltpu.VMEM((DSP_CHUNK,), jnp.int32),
            pltpu.VMEM((DSP_CHUNK,), jnp.int32),
            pltpu.VMEM((16,), jnp.int32),
            pltpu.VMEM((DSP_W * 16,), jnp.int32),
            pltpu.VMEM((DSP_SLICE,), jnp.int32),
            pltpu.VMEM((DSP_SLICE,), jnp.float32),
            pltpu.VMEM((NT_PAD,), jnp.int32),
            pltpu.VMEM_SHARED((DSP_W * 16,), jnp.int32),
            pltpu.VMEM_SHARED((A_PAD,), jnp.int32),
            pltpu.VMEM_SHARED((A_PAD,), jnp.float32),
        ],
    )(idx_flat, w_flat)


def _gather_body(x_hbm, src_hbm, xs_hbm, idxbuf, rowbuf, sem):
    wid = lax.axis_index("s") * 2 + lax.axis_index("c")

    def gbody(g, _):
        base = wid * G_ROWS + g * G_CHUNK
        pltpu.sync_copy(src_hbm.at[pl.ds(base, G_CHUNK)], idxbuf)
        pltpu.async_copy(x_hbm.at[idxbuf], rowbuf, sem).wait()
        pltpu.sync_copy(rowbuf, xs_hbm.at[pl.ds(base, G_CHUNK)])
        return 0
    lax.fori_loop(0, G_ROWS // G_CHUNK, gbody, 0)


def _gather(x, src_token):
    mesh = plsc.VectorSubcoreMesh(core_axis_name="c", subcore_axis_name="s")
    return pl.kernel(
        _gather_body,
        out_type=jax.ShapeDtypeStruct((A_PAD, D), jnp.float32),
        mesh=mesh,
        scratch_types=[
            pltpu.VMEM((G_CHUNK,), jnp.int32),
            pltpu.VMEM((G_CHUNK, D), jnp.float32),
            pltpu.SemaphoreType.DMA,
        ],
    )(x, src_token)


def _ffn_body(te_ref, x_ref, wg_ref, wu_ref, wd_ref, ws_ref, out_ref, acc):
    f = pl.program_id(1)
    x = x_ref[...]
    g = lax.dot_general(x, wg_ref[0], (((1,), (1,)), ((), ())),
                        preferred_element_type=jnp.float32)
    u = lax.dot_general(x, wu_ref[0], (((1,), (1,)), ((), ())),
                        preferred_element_type=jnp.float32)
    h = (g / (1.0 + jnp.exp(-g))) * u
    dpart = lax.dot_general(h, wd_ref[0], (((1,), (1,)), ((), ())),
                            preferred_element_type=jnp.float32)

    @pl.when(f == 0)
    def _():
        acc[...] = jnp.zeros_like(acc)

    acc[...] += dpart

    @pl.when(f == NF - 1)
    def _():
        out_ref[...] = acc[...] * ws_ref[0, 0].reshape(T, 1)


def _ffn(x_sorted, w_sorted_3d, tile_expert, w_gate, w_up, w_down):
    grid_spec = pltpu.PrefetchScalarGridSpec(
        num_scalar_prefetch=1,
        grid=(NT, NF),
        in_specs=[
            pl.BlockSpec((T, D), lambda t, f, te: (t, 0)),
            pl.BlockSpec((1, FFC, D), lambda t, f, te: (te[t], f, 0)),
            pl.BlockSpec((1, FFC, D), lambda t, f, te: (te[t], f, 0)),
            pl.BlockSpec((1, D, FFC), lambda t, f, te: (te[t], 0, f)),
            pl.BlockSpec((1, 1, T), lambda t, f, te: (t, 0, 0)),
        ],
        out_specs=pl.BlockSpec((T, D), lambda t, f, te: (t, 0)),
        scratch_shapes=[pltpu.VMEM((T, D), jnp.float32)],
    )
    return pl.pallas_call(
        _ffn_body,
        grid_spec=grid_spec,
        out_shape=jax.ShapeDtypeStruct((A_PAD, D), jnp.float32),
        compiler_params=pltpu.CompilerParams(
            dimension_semantics=("arbitrary", "arbitrary")),
    )(tile_expert, x_sorted, w_gate, w_up, w_down, w_sorted_3d)


def _combine_body(dw_hbm, pos_hbm, out_hbm, idxbuf, rowbuf, obuf, sem):
    wid = lax.axis_index("s") * 2 + lax.axis_index("c")

    def cbody(g, _):
        tb = wid * C_TOK + g * C_CHUNK
        pltpu.sync_copy(pos_hbm.at[pl.ds(2 * tb, 2 * C_CHUNK)], idxbuf)
        pltpu.async_copy(dw_hbm.at[idxbuf], rowbuf, sem).wait()

        def abody(j, _):
            for c in range(D // 16):
                sl = pl.ds(c * 16, 16)
                obuf[j, sl] = rowbuf[2 * j, sl] + rowbuf[2 * j + 1, sl]
            return 0
        lax.fori_loop(0, C_CHUNK, abody, 0)
        pltpu.sync_copy(obuf, out_hbm.at[pl.ds(tb, C_CHUNK)])
        return 0
    lax.fori_loop(0, C_TOK // C_CHUNK, cbody, 0)


def _combine(down_w, pos):
    mesh = plsc.VectorSubcoreMesh(core_axis_name="c", subcore_axis_name="s")
    return pl.kernel(
        _combine_body,
        out_type=jax.ShapeDtypeStruct((N_TOK, D), jnp.float32),
        mesh=mesh,
        scratch_types=[
            pltpu.VMEM((2 * C_CHUNK,), jnp.int32),
            pltpu.VMEM((2 * C_CHUNK, D), jnp.float32),
            pltpu.VMEM((C_CHUNK, D), jnp.float32),
            pltpu.SemaphoreType.DMA,
        ],
    )(down_w, pos)


@jax.jit
def kernel(hidden_states, W_router, W_gate, W_up, W_down):
    b, s, d = hidden_states.shape
    x = hidden_states.reshape(-1, d)
    topk_idx, topk_w, loss = _router(x, W_router)
    pos, src_token, w_sorted, tile_expert = _dispatch(
        topk_idx.reshape(-1), topk_w.reshape(-1))
    x_sorted = _gather(x, src_token)
    down_w = _ffn(x_sorted, w_sorted.reshape(NT, 1, T), tile_expert,
                  W_gate, W_up, W_down)
    out = _combine(down_w, pos)
    return out.reshape(b, s, d), loss[0, 0]


# speedup vs baseline: 1.8297x; 1.8297x over previous
import functools

import jax
import jax.numpy as jnp
from jax import lax
from jax.experimental import pallas as pl
from jax.experimental.pallas import tpu as pltpu
from jax.experimental.pallas import tpu_sc as plsc

D = 1024
FF = 4096
E = 8
K = 2
AUX_COEF = 0.01
Z_COEF = 0.001

N_TOK = 8192
A = N_TOK * K
T = 512
NT = A // T + E
NT_PAD = 48
A_PAD = NT * T

TB = 1024
FFC = 512
NF = FF // FFC

DSP_W = 16
DSP_CHUNK = A // DSP_W
DSP_VECS = DSP_CHUNK // 16
DSP_SLICE = A_PAD // DSP_W

GW = 32
G_ROWS = A_PAD // GW
G_CHUNK = 64
C_TOK = N_TOK // GW
C_CHUNK = 32


def _router_body(x_ref, wr_ref, idx_ref, w_ref, loss_ref, accp, accc, acclse):
    i = pl.program_id(0)
    x = x_ref[...]
    logits = lax.dot_general(x, wr_ref[...], (((1,), (1,)), ((), ())),
                             preferred_element_type=jnp.float32)
    eidx = lax.broadcasted_iota(jnp.int32, logits.shape, 1)
    m0 = jnp.max(logits, axis=-1, keepdims=True)
    i0 = jnp.min(jnp.where(logits == m0, eidx, E), axis=-1, keepdims=True)
    masked = jnp.where(eidx == i0, -jnp.inf, logits)
    m1 = jnp.max(masked, axis=-1, keepdims=True)
    i1 = jnp.min(jnp.where(masked == m1, eidx, E), axis=-1, keepdims=True)
    e1 = jnp.exp(m1 - m0)
    denom = 1.0 + e1
    idx_ref[...] = jnp.concatenate([i0, i1], axis=1)
    w_ref[...] = jnp.concatenate([1.0 / denom, e1 / denom], axis=1)
    ex = jnp.exp(logits - m0)
    sex = jnp.sum(ex, axis=-1, keepdims=True)
    probs = ex / sex
    lse = m0 + jnp.log(sex)

    @pl.when(i == 0)
    def _():
        accp[...] = jnp.zeros_like(accp)
        accc[...] = jnp.zeros_like(accc)
        acclse[...] = jnp.zeros_like(acclse)

    accp[...] += jnp.sum(probs, axis=0, keepdims=True)
    onehot = jnp.where(eidx == i0, 1.0, 0.0) + jnp.where(eidx == i1, 1.0, 0.0)
    accc[...] += jnp.sum(onehot, axis=0, keepdims=True)
    acclse[...] += jnp.sum(lse).reshape(1, 1)

    @pl.when(i == pl.num_programs(0) - 1)
    def _():
        tokens_per_expert = accc[...] / (N_TOK * K)
        router_prob = accp[...] / N_TOK
        aux = E * jnp.sum(tokens_per_expert * router_prob) * AUX_COEF
        z = jnp.sum(acclse[...]) / N_TOK * Z_COEF
        loss_ref[...] = (aux + z).reshape(1, 1)


def _router(x, w_router):
    return pl.pallas_call(
        _router_body,
        grid=(N_TOK // TB,),
        in_specs=[
            pl.BlockSpec((TB, D), lambda i: (i, 0)),
            pl.BlockSpec((E, D), lambda i: (0, 0)),
        ],
        out_specs=[
            pl.BlockSpec((TB, K), lambda i: (i, 0)),
            pl.BlockSpec((TB, K), lambda i: (i, 0)),
            pl.BlockSpec((1, 1), lambda i: (0, 0)),
        ],
        out_shape=[
            jax.ShapeDtypeStruct((N_TOK, K), jnp.int32),
            jax.ShapeDtypeStruct((N_TOK, K), jnp.float32),
            jax.ShapeDtypeStruct((1, 1), jnp.float32),
        ],
        scratch_shapes=[
            pltpu.VMEM((1, E), jnp.float32),
            pltpu.VMEM((1, E), jnp.float32),
            pltpu.VMEM((1, 1), jnp.float32),
        ],
    )(x, w_router)


def _dispatch_body(idx_hbm, w_hbm, pos_hbm, src_hbm, wsort_hbm, te_hbm,
                   idx_c, w_c, pbuf, tbuf, cnt_ref, hist_local, zb_i, zb_f,
                   tebuf, sh_hist, sh_src, sh_w):
    wid = lax.axis_index("s")
    lane = lax.broadcasted_iota(jnp.int32, (16,), 0)
    zi16 = jnp.zeros((16,), jnp.int32)

    def zbody(j, _):
        zb_i[pl.ds(j * 16, 16)] = zi16
        zb_f[pl.ds(j * 16, 16)] = jnp.zeros((16,), jnp.float32)
        return 0
    lax.fori_loop(0, DSP_SLICE // 16, zbody, 0)
    pltpu.sync_copy(zb_i, sh_src.at[pl.ds(wid * DSP_SLICE, DSP_SLICE)])
    pltpu.sync_copy(zb_f, sh_w.at[pl.ds(wid * DSP_SLICE, DSP_SLICE)])

    pltpu.sync_copy(idx_hbm.at[pl.ds(wid * DSP_CHUNK, DSP_CHUNK)], idx_c)
    pltpu.sync_copy(w_hbm.at[pl.ds(wid * DSP_CHUNK, DSP_CHUNK)], w_c)

    def hbody(j, cnt):
        v = idx_c[pl.ds(j * 16, 16)]
        for u in range(16):
            cnt = cnt + jnp.where(lane == v[u], 1, 0)
        return cnt
    local_cnt = lax.fori_loop(0, DSP_VECS, hbody, zi16)
    cnt_ref[pl.ds(0, 16)] = local_cnt
    pltpu.sync_copy(cnt_ref.at[pl.ds(0, 16)], sh_hist.at[pl.ds(wid * 16, 16)])
    plsc.subcore_barrier()

    pltpu.sync_copy(sh_hist, hist_local)

    def rbody(w, carry):
        g, mb = carry
        row = hist_local[pl.ds(w * 16, 16)]
        return g + row, mb + jnp.where(w < wid, row, 0)
    gcnt, my_base = lax.fori_loop(0, DSP_W, rbody, (zi16, zi16))

    padded = (gcnt + (T - 1)) & (-T)
    offs = zi16
    run = jnp.int32(0)
    for e in range(E):
        offs = offs + jnp.where(lane == e, run, 0)
        run = run + padded[e]
    base0 = offs + my_base

    cnt_ref[pl.ds(0, 16)] = base0

    def pbody(j, cnt):
        off = j * 16
        v = idx_c[pl.ds(off, 16)]
        pv = zi16
        newcnt = cnt
        for u in range(16):
            ev = v[u]
            win = cnt_ref[pl.ds(ev, 16)]
            pv = pv + jnp.where(lane == u, win[0], 0)
            newcnt = newcnt + jnp.where(lane == ev, 1, 0)
            cnt_ref[pl.ds(0, 16)] = newcnt
        pbuf[pl.ds(off, 16)] = pv
        tbuf[pl.ds(off, 16)] = lax.shift_right_logical(
            wid * DSP_CHUNK + off + lane, 1)
        return newcnt
    lax.fori_loop(0, DSP_VECS, pbody, base0)

    pltpu.sync_copy(pbuf, pos_hbm.at[pl.ds(wid * DSP_CHUNK, DSP_CHUNK)])

    def sbody(j, _):
        off = j * 16
        idxv = pbuf[pl.ds(off, 16)]
        pltpu.sync_copy(tbuf.at[pl.ds(off, 16)], sh_src.at[idxv])
        pltpu.sync_copy(w_c.at[pl.ds(off, 16)], sh_w.at[idxv])
        return 0
    lax.fori_loop(0, DSP_VECS, sbody, 0)
    plsc.subcore_barrier()

    sl = pl.ds(wid * DSP_SLICE, DSP_SLICE)
    pltpu.sync_copy(sh_src.at[sl], zb_i)
    pltpu.sync_copy(zb_i, src_hbm.at[sl])
    pltpu.sync_copy(sh_w.at[sl], zb_f)
    pltpu.sync_copy(zb_f, wsort_hbm.at[sl])

    @pl.when(wid == 0)
    def _():
        ends = offs + padded
        for c in range(NT_PAD // 16):
            tid = c * 16 + lane
            acc = zi16
            for e in range(E):
                acc = acc + jnp.where(tid * T >= ends[e], 1, 0)
            tebuf[pl.ds(c * 16, 16)] = jnp.minimum(acc, E - 1)
        pltpu.sync_copy(tebuf, te_hbm)


def _dispatch(idx_flat, w_flat):
    mesh = plsc.VectorSubcoreMesh(core_axis_name="c", subcore_axis_name="s",
                                  num_cores=1, num_subcores=DSP_W)
    return pl.kernel(
        _dispatch_body,
        out_type=[
            jax.ShapeDtypeStruct((A,), jnp.int32),
            jax.ShapeDtypeStruct((A_PAD,), jnp.int32),
            jax.ShapeDtypeStruct((A_PAD,), jnp.float32),
            jax.ShapeDtypeStruct((NT_PAD,), jnp.int32),
        ],
        mesh=mesh,
        scratch_types=[
            pltpu.VMEM((DSP_CHUNK,), jnp.int32),
            pltpu.VMEM((DSP_CHUNK,), jnp.float32),
            pltpu.VMEM((DSP_CHUNK,), jnp.int32),
            pltpu.VMEM((DSP_CHUNK,), jnp.int32),
            pltpu.VMEM((32,), jnp.int32),
            pltpu.VMEM((DSP_W * 16,), jnp.int32),
            pltpu.VMEM((DSP_SLICE,), jnp.int32),
            pltpu.VMEM((DSP_SLICE,), jnp.float32),
            pltpu.VMEM((NT_PAD,), jnp.int32),
            pltpu.VMEM_SHARED((DSP_W * 16,), jnp.int32),
            pltpu.VMEM_SHARED((A_PAD,), jnp.int32),
            pltpu.VMEM_SHARED((A_PAD,), jnp.float32),
        ],
    )(idx_flat, w_flat)


def _gather_body(x_hbm, src_hbm, xs_hbm, idxbuf, rowbuf, sem):
    wid = lax.axis_index("s") * 2 + lax.axis_index("c")

    def gbody(g, _):
        base = wid * G_ROWS + g * G_CHUNK
        pltpu.sync_copy(src_hbm.at[pl.ds(base, G_CHUNK)], idxbuf)
        pltpu.async_copy(x_hbm.at[idxbuf], rowbuf, sem).wait()
        pltpu.sync_copy(rowbuf, xs_hbm.at[pl.ds(base, G_CHUNK)])
        return 0
    lax.fori_loop(0, G_ROWS // G_CHUNK, gbody, 0)


def _gather(x, src_token):
    mesh = plsc.VectorSubcoreMesh(core_axis_name="c", subcore_axis_name="s",
                                  num_cores=2, num_subcores=16)
    return pl.kernel(
        _gather_body,
        out_type=jax.ShapeDtypeStruct((A_PAD, D), jnp.float32),
        mesh=mesh,
        scratch_types=[
            pltpu.VMEM((G_CHUNK,), jnp.int32),
            pltpu.VMEM((G_CHUNK, D), jnp.float32),
            pltpu.SemaphoreType.DMA,
        ],
    )(x, src_token)


def _ffn_body(te_ref, x_ref, wg_ref, wu_ref, wd_ref, ws_ref, out_ref, acc):
    f = pl.program_id(1)
    x = x_ref[...]
    g = lax.dot_general(x, wg_ref[0], (((1,), (1,)), ((), ())),
                        preferred_element_type=jnp.float32)
    u = lax.dot_general(x, wu_ref[0], (((1,), (1,)), ((), ())),
                        preferred_element_type=jnp.float32)
    h = (g / (1.0 + jnp.exp(-g))) * u
    dpart = lax.dot_general(h, wd_ref[0], (((1,), (1,)), ((), ())),
                            preferred_element_type=jnp.float32)

    @pl.when(f == 0)
    def _():
        acc[...] = jnp.zeros_like(acc)

    acc[...] += dpart

    @pl.when(f == NF - 1)
    def _():
        out_ref[...] = acc[...] * ws_ref[0, 0].reshape(T, 1)


def _ffn(x_sorted, w_sorted_3d, tile_expert, w_gate, w_up, w_down):
    grid_spec = pltpu.PrefetchScalarGridSpec(
        num_scalar_prefetch=1,
        grid=(NT, NF),
        in_specs=[
            pl.BlockSpec((T, D), lambda t, f, te: (t, 0)),
            pl.BlockSpec((1, FFC, D), lambda t, f, te: (te[t], f, 0)),
            pl.BlockSpec((1, FFC, D), lambda t, f, te: (te[t], f, 0)),
            pl.BlockSpec((1, D, FFC), lambda t, f, te: (te[t], 0, f)),
            pl.BlockSpec((1, 1, T), lambda t, f, te: (t, 0, 0)),
        ],
        out_specs=pl.BlockSpec((T, D), lambda t, f, te: (t, 0)),
        scratch_shapes=[pltpu.VMEM((T, D), jnp.float32)],
    )
    return pl.pallas_call(
        _ffn_body,
        grid_spec=grid_spec,
        out_shape=jax.ShapeDtypeStruct((A_PAD, D), jnp.float32),
        compiler_params=pltpu.CompilerParams(
            dimension_semantics=("arbitrary", "arbitrary")),
    )(tile_expert, x_sorted, w_gate, w_up, w_down, w_sorted_3d)


def _combine_body(dw_hbm, pos_hbm, out_hbm, idxbuf, rowbuf, obuf, sem):
    wid = lax.axis_index("s") * 2 + lax.axis_index("c")

    def cbody(g, _):
        tb = wid * C_TOK + g * C_CHUNK
        pltpu.sync_copy(pos_hbm.at[pl.ds(2 * tb, 2 * C_CHUNK)], idxbuf)
        pltpu.async_copy(dw_hbm.at[idxbuf], rowbuf, sem).wait()

        def abody(j, _):
            for c in range(D // 16):
                sl = pl.ds(c * 16, 16)
                obuf[j, sl] = rowbuf[2 * j, sl] + rowbuf[2 * j + 1, sl]
            return 0
        lax.fori_loop(0, C_CHUNK, abody, 0)
        pltpu.sync_copy(obuf, out_hbm.at[pl.ds(tb, C_CHUNK)])
        return 0
    lax.fori_loop(0, C_TOK // C_CHUNK, cbody, 0)


def _combine(down_w, pos):
    mesh = plsc.VectorSubcoreMesh(core_axis_name="c", subcore_axis_name="s",
                                  num_cores=2, num_subcores=16)
    return pl.kernel(
        _combine_body,
        out_type=jax.ShapeDtypeStruct((N_TOK, D), jnp.float32),
        mesh=mesh,
        scratch_types=[
            pltpu.VMEM((2 * C_CHUNK,), jnp.int32),
            pltpu.VMEM((2 * C_CHUNK, D), jnp.float32),
            pltpu.VMEM((C_CHUNK, D), jnp.float32),
            pltpu.SemaphoreType.DMA,
        ],
    )(down_w, pos)


@jax.jit
def kernel(hidden_states, W_router, W_gate, W_up, W_down):
    b, s, d = hidden_states.shape
    x = hidden_states.reshape(-1, d)
    topk_idx, topk_w, loss = _router(x, W_router)
    pos, src_token, w_sorted, tile_expert = _dispatch(
        topk_idx.reshape(-1), topk_w.reshape(-1))
    x_sorted = _gather(x, src_token)
    down_w = _ffn(x_sorted, w_sorted.reshape(NT, 1, T), tile_expert,
                  W_gate, W_up, W_down)
    out = _combine(down_w, pos)
    return out.reshape(b, s, d), loss[0, 0]
